# baseline (speedup 1.0000x reference)
import functools

import jax
import jax.numpy as jnp
from jax import lax
from jax.experimental import pallas as pl
from jax.experimental.pallas import tpu as pltpu
from jax.experimental.pallas import tpu_sc as plsc

_K = 16
_BR = 256
_L = 16
_T = 8
_NU = 1152
_T2 = 9
_NU2 = 128


def _dist_body(q_ref, k_ref, d_ref, a_ref, a2_ref, ct_ref, *, n, k):
    b = pl.program_id(0)
    q = q_ref[0]
    kt = k_ref[0]
    c_dim = q.shape[1]

    qs = q[:, 0:1] * q[:, 0:1]
    for c in range(1, c_dim):
        qs = qs + q[:, c:c + 1] * q[:, c:c + 1]
    qn = q / jnp.maximum(jnp.sqrt(qs), 1e-12)

    ks = kt[0:1] * kt[0:1]
    for c in range(1, c_dim):
        ks = ks + kt[c:c + 1] * kt[c:c + 1]
    knt = kt / jnp.maximum(jnp.sqrt(ks), 1e-12)

    dots = lax.dot_general(qn, knt, (((1,), (0,)), ((), ())),
                           preferred_element_type=jnp.float32)

    sq_q = qn[:, 0:1] * qn[:, 0:1]
    for c in range(1, c_dim):
        sq_q = sq_q + qn[:, c:c + 1] * qn[:, c:c + 1]
    sq_k = knt[0:1] * knt[0:1]
    for c in range(1, c_dim):
        sq_k = sq_k + knt[c:c + 1] * knt[c:c + 1]

    nd = -((sq_q + (-2.0 * dots)) + sq_k)
    d_ref[0] = nd

    acc = nd[:, 0:_NU]
    for t in range(1, _T):
        acc = jnp.maximum(acc, nd[:, t * _NU:(t + 1) * _NU])
    a_ref[0] = acc

    acc2 = acc[:, 0:_NU2]
    for t in range(1, _T2):
        acc2 = jnp.maximum(acc2, acc[:, t * _NU2:(t + 1) * _NU2])
    a2_ref[0] = acc2

    row0 = b * n + pl.program_id(1) * _BR
    ct_ref[0] = lax.broadcasted_iota(jnp.int32, (_BR, k), 0) + row0


def _topk_sc_body(dist_hbm, accm_hbm, acc2m_hbm, nn_hbm, buf2, accb2,
                  acc2b2, outv, semd, sema, sem2, *, n, nrows, k):
    nc = 2
    wid = lax.axis_index("s") * nc + lax.axis_index("c")
    rpw = nrows // 32
    base = wid * rpw
    lanes = jnp.arange(_L, dtype=jnp.int32)
    tcap = jnp.minimum(lanes, _T - 1)
    t2cap = jnp.minimum(lanes, _T2 - 1)
    gpw = rpw // 4
    gbase = wid * gpw
    glast = gbase + gpw - 1
    ninf = jnp.float32(-jnp.inf)

    def start(b0, b1, b2, s0, s1, s2, half, grp):
        pltpu.make_async_copy(dist_hbm.at[grp],
                              b0.at[pl.ds(half * 4 * n, 4 * n)], s0).start()
        pltpu.make_async_copy(accm_hbm.at[grp],
                              b1.at[pl.ds(half * 4 * _NU, 4 * _NU)],
                              s1).start()
        pltpu.make_async_copy(acc2m_hbm.at[grp],
                              b2.at[pl.ds(half * 4 * _NU2, 4 * _NU2)],
                              s2).start()

    def wait(b0, b1, b2, s0, s1, s2):
        pltpu.make_async_copy(dist_hbm.at[gbase],
                              b0.at[pl.ds(0, 4 * n)], s0).wait()
        pltpu.make_async_copy(accm_hbm.at[gbase],
                              b1.at[pl.ds(0, 4 * _NU)], s1).wait()
        pltpu.make_async_copy(acc2m_hbm.at[gbase],
                              b2.at[pl.ds(0, 4 * _NU2)], s2).wait()

    def process(h, rA, rB, rowA, rowB):

        def one(buf, accb, acc2b, bb, ab, a2b, t, idxs):
            cur = acc2b[pl.ds(a2b, _L)]
            jsel = jnp.zeros((_L,), jnp.int32)
            for a in range(1, _NU2 // _L):
                v = acc2b[pl.ds(a2b + a * _L, _L)]
                gt = v > cur
                cur = jnp.where(gt, v, cur)
                jsel = jnp.where(gt, a, jsel)
            sk, sv = plsc.sort_key_val(cur, jsel * _L + lanes, descending=True)
            m = sk[0]
            w = sv[0]
            lstar = jnp.bitwise_and(w, _L - 1)
            g1 = plsc.load_gather(accb, [ab + t2cap * _NU2 + w])
            _, s1 = plsc.sort_key_val(g1, lanes, descending=True)
            u = s1[0] * _NU2 + w
            g0 = plsc.load_gather(buf, [bb + tcap * _NU + u])
            _, s0 = plsc.sort_key_val(g0, lanes, descending=True)
            col = s0[0] * _NU + u
            vv = buf[pl.ds(bb + col - lstar, _L)]
            buf[pl.ds(bb + col - lstar, _L)] = jnp.where(
                lanes == lstar, ninf, vv)
            r0 = buf[pl.ds(bb + u - lstar, _L)]
            for tt in range(1, _T):
                r0 = jnp.maximum(r0, buf[pl.ds(bb + tt * _NU + u - lstar, _L)])
            accb[pl.ds(ab + u - lstar, _L)] = r0
            r1 = accb[pl.ds(ab + w - lstar, _L)]
            for tt in range(1, _T2):
                r1 = jnp.maximum(
                    r1, accb[pl.ds(ab + tt * _NU2 + w - lstar, _L)])
            acc2b[pl.ds(a2b + w - lstar, _L)] = r1
            return jnp.where(lanes == (t - 1), col, idxs)

        bbA = (h * 4 + rA) * n
        abA = (h * 4 + rA) * _NU
        a2bA = (h * 4 + rA) * _NU2
        bbB = (h * 4 + rB) * n
        abB = (h * 4 + rB) * _NU
        a2bB = (h * 4 + rB) * _NU2

        def round_t(t, carry):
            idxsA, idxsB = carry
            idxsA = one(buf2, accb2, acc2b2, bbA, abA, a2bA, t, idxsA)
            idxsB = one(buf2, accb2, acc2b2, bbB, abB, a2bB, t, idxsB)
            return idxsA, idxsB

        idxsA, idxsB = lax.fori_loop(
            0, k + 1, round_t,
            (jnp.zeros((_L,), jnp.int32), jnp.zeros((_L,), jnp.int32)))

        offA = jnp.where(rowA >= n, n, 0)
        offB = jnp.where(rowB >= n, n, 0)
        outv[pl.ds(rA * _L, _L)] = idxsA + offA
        outv[pl.ds(rB * _L, _L)] = idxsB + offB

    start(buf2, accb2, acc2b2, semd, sema, sem2, 0, gbase)

    def outer(g, carry):
        p = jnp.bitwise_and(g, 1)
        grp = gbase + g
        row4 = 4 * grp
        wait(buf2, accb2, acc2b2, semd, sema, sem2)
        start(buf2, accb2, acc2b2, semd, sema, sem2, 1 - p,
              jnp.minimum(grp + 1, glast))
        process(p, 0, 1, row4, row4 + 1)
        process(p, 2, 3, row4 + 2, row4 + 3)
        pltpu.sync_copy(outv, nn_hbm.at[grp])
        return carry
    lax.fori_loop(0, gpw, outer, 0)
    wait(buf2, accb2, acc2b2, semd, sema, sem2)


def kernel(x):
    B, C, H, W = x.shape
    n = H * W
    rows = B * n
    xc = x.reshape(B, C, n)
    xt = jnp.transpose(xc, (0, 2, 1))

    dist, accm, acc2m, ct = pl.pallas_call(
        functools.partial(_dist_body, n=n, k=_K),
        grid=(B, n // _BR),
        in_specs=[
            pl.BlockSpec((1, _BR, C), lambda b, r: (b, r, 0)),
            pl.BlockSpec((1, C, n), lambda b, r: (b, 0, 0)),
        ],
        out_specs=[
            pl.BlockSpec((1, _BR, n), lambda b, r: (b, r, 0)),
            pl.BlockSpec((1, _BR, _NU), lambda b, r: (b, r, 0)),
            pl.BlockSpec((1, _BR, _NU2), lambda b, r: (b, r, 0)),
            pl.BlockSpec((1, _BR, _K), lambda b, r: (b, r, 0)),
        ],
        out_shape=[
            jax.ShapeDtypeStruct((B, n, n), jnp.float32),
            jax.ShapeDtypeStruct((B, n, _NU), jnp.float32),
            jax.ShapeDtypeStruct((B, n, _NU2), jnp.float32),
            jax.ShapeDtypeStruct((B, n, _K), jnp.int32),
        ],
    )(xt, xc)

    mesh = plsc.VectorSubcoreMesh(core_axis_name="c", subcore_axis_name="s")
    nn = pl.kernel(
        functools.partial(_topk_sc_body, n=n, nrows=rows, k=_K),
        out_type=jax.ShapeDtypeStruct((rows // 4, 4 * _K), jnp.int32),
        mesh=mesh,
        compiler_params=pltpu.CompilerParams(needs_layout_passes=False),
        scratch_types=[
            pltpu.VMEM((2 * 4 * n,), jnp.float32),
            pltpu.VMEM((2 * 4 * _NU,), jnp.float32),
            pltpu.VMEM((2 * 4 * _NU2,), jnp.float32),
            pltpu.VMEM((4 * _L,), jnp.int32),
            pltpu.SemaphoreType.DMA,
            pltpu.SemaphoreType.DMA,
            pltpu.SemaphoreType.DMA,
        ],
    )(dist.reshape(rows // 4, 4 * n), accm.reshape(rows // 4, 4 * _NU),
      acc2m.reshape(rows // 4, 4 * _NU2))
    return jnp.stack((nn.reshape(-1), ct.reshape(-1)), axis=0)


# speedup vs baseline: 1.5471x; 1.5471x over previous
import functools

import jax
import jax.numpy as jnp
from jax import lax
from jax.experimental import pallas as pl
from jax.experimental.pallas import tpu as pltpu
from jax.experimental.pallas import tpu_sc as plsc

_K = 16
_BR = 256
_L = 16
_T = 8
_NU = 1152
_T2 = 9
_NU2 = 128


def _dist_body(q_ref, k_ref, d_ref, a_ref, a2_ref, ct_ref, *, n, k):
    b = pl.program_id(0)
    q = q_ref[0]
    kt = k_ref[0]
    c_dim = q.shape[1]

    qs = q[:, 0:1] * q[:, 0:1]
    for c in range(1, c_dim):
        qs = qs + q[:, c:c + 1] * q[:, c:c + 1]
    qn = q / jnp.maximum(jnp.sqrt(qs), 1e-12)

    ks = kt[0:1] * kt[0:1]
    for c in range(1, c_dim):
        ks = ks + kt[c:c + 1] * kt[c:c + 1]
    knt = kt / jnp.maximum(jnp.sqrt(ks), 1e-12)

    dots = lax.dot_general(qn, knt, (((1,), (0,)), ((), ())),
                           preferred_element_type=jnp.float32)

    sq_q = qn[:, 0:1] * qn[:, 0:1]
    for c in range(1, c_dim):
        sq_q = sq_q + qn[:, c:c + 1] * qn[:, c:c + 1]
    sq_k = knt[0:1] * knt[0:1]
    for c in range(1, c_dim):
        sq_k = sq_k + knt[c:c + 1] * knt[c:c + 1]

    nd = -((sq_q + (-2.0 * dots)) + sq_k)
    d_ref[0] = nd

    acc = nd[:, 0:_NU]
    for t in range(1, _T):
        acc = jnp.maximum(acc, nd[:, t * _NU:(t + 1) * _NU])
    a_ref[0] = acc

    acc2 = acc[:, 0:_NU2]
    for t in range(1, _T2):
        acc2 = jnp.maximum(acc2, acc[:, t * _NU2:(t + 1) * _NU2])
    a2_ref[0] = acc2

    row0 = b * n + pl.program_id(1) * _BR
    ct_ref[0] = lax.broadcasted_iota(jnp.int32, (_BR, k), 0) + row0


def _topk_sc_body(dist_hbm, accm_hbm, acc2m_hbm, nn_hbm, buf, accb, acc2b,
                  outv, bufB, accbB, acc2bB, outvB, semd, sema, sem2,
                  semdB, semaB, sem2B, *, n, nrows, k):
    nc = 2
    wid = lax.axis_index("s") * nc + lax.axis_index("c")
    rpw = nrows // 32
    base = wid * rpw
    lanes = jnp.arange(_L, dtype=jnp.int32)
    tcap = jnp.minimum(lanes, _T - 1)
    t2cap = jnp.minimum(lanes, _T2 - 1)
    last = base + rpw - 1
    ninf = jnp.float32(-jnp.inf)

    def start(b0, b1, b2, s0, s1, s2, half, row):
        pltpu.make_async_copy(dist_hbm.at[row], b0.at[pl.ds(half * n, n)],
                              s0).start()
        pltpu.make_async_copy(accm_hbm.at[row],
                              b1.at[pl.ds(half * _NU, _NU)], s1).start()
        pltpu.make_async_copy(acc2m_hbm.at[row],
                              b2.at[pl.ds(half * _NU2, _NU2)], s2).start()

    def wait(b0, b1, b2, s0, s1, s2):
        pltpu.make_async_copy(dist_hbm.at[base], b0.at[pl.ds(0, n)],
                              s0).wait()
        pltpu.make_async_copy(accm_hbm.at[base], b1.at[pl.ds(0, _NU)],
                              s1).wait()
        pltpu.make_async_copy(acc2m_hbm.at[base], b2.at[pl.ds(0, _NU2)],
                              s2).wait()

    def process(h, rowA, rowB):
        bb, ab, a2b = h * n, h * _NU, h * _NU2

        def one(buf, accb, acc2b, t, idxs):
            cur = acc2b[pl.ds(a2b, _L)]
            jsel = jnp.zeros((_L,), jnp.int32)
            for a in range(1, _NU2 // _L):
                v = acc2b[pl.ds(a2b + a * _L, _L)]
                gt = v > cur
                cur = jnp.where(gt, v, cur)
                jsel = jnp.where(gt, a, jsel)
            sk, sv = plsc.sort_key_val(cur, jsel * _L + lanes, descending=True)
            m = sk[0]
            w = sv[0]
            lstar = jnp.bitwise_and(w, _L - 1)
            g1 = plsc.load_gather(accb, [ab + t2cap * _NU2 + w])
            _, s1 = plsc.sort_key_val(g1, lanes, descending=True)
            u = s1[0] * _NU2 + w
            g0 = plsc.load_gather(buf, [bb + tcap * _NU + u])
            _, s0 = plsc.sort_key_val(g0, lanes, descending=True)
            col = s0[0] * _NU + u
            vv = buf[pl.ds(bb + col - lstar, _L)]
            buf[pl.ds(bb + col - lstar, _L)] = jnp.where(
                lanes == lstar, ninf, vv)
            r0 = buf[pl.ds(bb + u - lstar, _L)]
            for tt in range(1, _T):
                r0 = jnp.maximum(r0, buf[pl.ds(bb + tt * _NU + u - lstar, _L)])
            accb[pl.ds(ab + u - lstar, _L)] = r0
            r1 = accb[pl.ds(ab + w - lstar, _L)]
            for tt in range(1, _T2):
                r1 = jnp.maximum(
                    r1, accb[pl.ds(ab + tt * _NU2 + w - lstar, _L)])
            acc2b[pl.ds(a2b + w - lstar, _L)] = r1
            return jnp.where(lanes == (t - 1), col, idxs)

        def round_t(t, carry):
            idxsA, idxsB = carry
            idxsA = one(buf, accb, acc2b, t, idxsA)
            idxsB = one(bufB, accbB, acc2bB, t, idxsB)
            return idxsA, idxsB

        idxsA, idxsB = lax.fori_loop(
            0, k + 1, round_t,
            (jnp.zeros((_L,), jnp.int32), jnp.zeros((_L,), jnp.int32)))

        offA = jnp.where(rowA >= n, n, 0)
        offB = jnp.where(rowB >= n, n, 0)
        outv[...] = idxsA + offA
        outvB[...] = idxsB + offB
        pltpu.sync_copy(outv, nn_hbm.at[rowA])
        pltpu.sync_copy(outvB, nn_hbm.at[rowB])

    start(buf, accb, acc2b, semd, sema, sem2, 0, base)
    start(bufB, accbB, acc2bB, semdB, semaB, sem2B, 0, base + 1)

    def outer(i, carry):
        p = jnp.bitwise_and(i, 1)
        rowA = base + 2 * i
        rowB = rowA + 1
        wait(buf, accb, acc2b, semd, sema, sem2)
        wait(bufB, accbB, acc2bB, semdB, semaB, sem2B)
        start(buf, accb, acc2b, semd, sema, sem2, 1 - p,
              jnp.minimum(rowA + 2, last))
        start(bufB, accbB, acc2bB, semdB, semaB, sem2B, 1 - p,
              jnp.minimum(rowB + 2, last))
        process(p, rowA, rowB)
        return carry
    lax.fori_loop(0, rpw // 2, outer, 0)
    wait(buf, accb, acc2b, semd, sema, sem2)
    wait(bufB, accbB, acc2bB, semdB, semaB, sem2B)


def kernel(x):
    B, C, H, W = x.shape
    n = H * W
    rows = B * n
    xc = x.reshape(B, C, n)
    xt = jnp.transpose(xc, (0, 2, 1))

    dist, accm, acc2m, ct = pl.pallas_call(
        functools.partial(_dist_body, n=n, k=_K),
        grid=(B, n // _BR),
        in_specs=[
            pl.BlockSpec((1, _BR, C), lambda b, r: (b, r, 0)),
            pl.BlockSpec((1, C, n), lambda b, r: (b, 0, 0)),
        ],
        out_specs=[
            pl.BlockSpec((1, _BR, n), lambda b, r: (b, r, 0)),
            pl.BlockSpec((1, _BR, _NU), lambda b, r: (b, r, 0)),
            pl.BlockSpec((1, _BR, _NU2), lambda b, r: (b, r, 0)),
            pl.BlockSpec((1, _BR, _K), lambda b, r: (b, r, 0)),
        ],
        out_shape=[
            jax.ShapeDtypeStruct((B, n, n), jnp.float32),
            jax.ShapeDtypeStruct((B, n, _NU), jnp.float32),
            jax.ShapeDtypeStruct((B, n, _NU2), jnp.float32),
            jax.ShapeDtypeStruct((B, n, _K), jnp.int32),
        ],
    )(xt, xc)

    mesh = plsc.VectorSubcoreMesh(core_axis_name="c", subcore_axis_name="s")
    nn = pl.kernel(
        functools.partial(_topk_sc_body, n=n, nrows=rows, k=_K),
        out_type=jax.ShapeDtypeStruct((rows, _K), jnp.int32),
        mesh=mesh,
        compiler_params=pltpu.CompilerParams(needs_layout_passes=False),
        scratch_types=[
            pltpu.VMEM((2 * n,), jnp.float32),
            pltpu.VMEM((2 * _NU,), jnp.float32),
            pltpu.VMEM((2 * _NU2,), jnp.float32),
            pltpu.VMEM((_L,), jnp.int32),
            pltpu.VMEM((2 * n,), jnp.float32),
            pltpu.VMEM((2 * _NU,), jnp.float32),
            pltpu.VMEM((2 * _NU2,), jnp.float32),
            pltpu.VMEM((_L,), jnp.int32),
            pltpu.SemaphoreType.DMA,
            pltpu.SemaphoreType.DMA,
            pltpu.SemaphoreType.DMA,
            pltpu.SemaphoreType.DMA,
            pltpu.SemaphoreType.DMA,
            pltpu.SemaphoreType.DMA,
        ],
    )(dist.reshape(rows, n), accm.reshape(rows, _NU),
      acc2m.reshape(rows, _NU2))
    return jnp.stack((nn.reshape(-1), ct.reshape(-1)), axis=0)
